# baseline (speedup 1.0000x reference)
import functools

import jax
import jax.numpy as jnp
from jax import lax
from jax.experimental import pallas as pl
from jax.experimental.pallas import tpu as pltpu
from jax.experimental.pallas import tpu_sc as plsc

N = 10000
NPAD = 10112
F = 128
NG = 64
NCLS = 10
EPS = 1e-5
E = 320000
NW = 32
CHUNK = 128
CPW = 80
EPAD = NW * CPW * CHUNK


def _bn(x, g, b):
    m = jnp.mean(x, axis=0, keepdims=True)
    v = jnp.mean((x - m) ** 2, axis=0, keepdims=True)
    return g * (x - m) * lax.rsqrt(v + EPS) + b


def _dis_from_parts(degp_ref):
    deg = degp_ref[0, :N, :1] + degp_ref[1, :N, :1]
    return lax.rsqrt(deg + 1.0)


def _tc_feat_body(x_ref, g_ref, b_ref, Wf_ref, g0_ref, b0_ref, Wc0_ref,
                  degp_ref, tab0_ref):
    h = _bn(x_ref[...], g_ref[...], b_ref[...])
    h = jnp.maximum(jnp.dot(h, Wf_ref[...], preferred_element_type=jnp.float32), 0.0)
    t0 = jnp.dot(_bn(h, g0_ref[...], b0_ref[...]), Wc0_ref[...],
                 preferred_element_type=jnp.float32)
    dis = _dis_from_parts(degp_ref)
    tab0_ref[:N, :] = dis * t0
    tab0_ref[N:, :] = jnp.zeros((NPAD - N, F), jnp.float32)


def _tc_combine_body(parts_ref, tab_ref, degp_ref, bc_ref, g_ref, b_ref,
                     W_ref, out_ref):
    dis = _dis_from_parts(degp_ref)
    S = parts_ref[0, :N, :] + parts_ref[1, :N, :]
    h = jnp.maximum(dis * (S + tab_ref[:N, :]) + bc_ref[...], 0.0)
    t = jnp.dot(_bn(h, g_ref[...], b_ref[...]), W_ref[...],
                preferred_element_type=jnp.float32)
    out_ref[:N, :] = dis * t
    out_ref[N:, :] = jnp.zeros((NPAD - N, F), jnp.float32)


def _tc_head_body(parts_ref, tab_ref, degp_ref, bc_ref, batch_ref,
                  fcg_ref, fcb_ref, Wfc_ref, bfc_ref, hg_ref, hb_ref,
                  Wcls_ref, bcls_ref, out_ref):
    dis = _dis_from_parts(degp_ref)
    S = parts_ref[0, :N, :] + parts_ref[1, :N, :]
    h = jnp.maximum(dis * (S + tab_ref[:N, :]) + bc_ref[...], 0.0)
    oh = (lax.broadcasted_iota(jnp.int32, (NG, N), 0) == batch_ref[...]
          ).astype(jnp.float32)
    pooled = jnp.dot(oh, h, preferred_element_type=jnp.float32)
    p = _bn(pooled, fcg_ref[...], fcb_ref[...])
    p = jnp.maximum(jnp.dot(p, Wfc_ref[...], preferred_element_type=jnp.float32)
                    + bfc_ref[...], 0.0)
    p = _bn(p, hg_ref[...], hb_ref[...])
    logits = jnp.dot(p, Wcls_ref[...], preferred_element_type=jnp.float32) + bcls_ref[...]
    m = jnp.max(logits, axis=-1, keepdims=True)
    lse = jnp.log(jnp.sum(jnp.exp(logits - m), axis=-1, keepdims=True))
    out_ref[...] = (logits - m) - lse


def _tc_call(body, out_shape, *args):
    return pl.pallas_call(body, out_shape=out_shape)(*args)


_MESH = plsc.VectorSubcoreMesh(core_axis_name="c", subcore_axis_name="s")
_NSUB = 16
_RPS = NPAD // _NSUB


@functools.partial(
    pl.kernel,
    out_type=jax.ShapeDtypeStruct((2, NPAD, F), jnp.float32),
    mesh=_MESH,
    scratch_types=[
        pltpu.VMEM((CPW, CHUNK), jnp.int32),
        pltpu.VMEM((CHUNK, F), jnp.float32),
        pltpu.VMEM_SHARED((NPAD, F), jnp.float32),
    ],
)
def _sc_deg(r_hbm, ones_hbm, zeros_hbm, out_hbm, ridx_v, ones_v, deg_sh):
    c = lax.axis_index("c")
    s = lax.axis_index("s")
    w = c * _NSUB + s
    base = s * _RPS
    pltpu.sync_copy(zeros_hbm.at[pl.ds(base, _RPS)], deg_sh.at[pl.ds(base, _RPS)])
    pltpu.sync_copy(r_hbm.at[pl.ds(pl.multiple_of(w * CPW, CPW), CPW)], ridx_v)
    pltpu.sync_copy(ones_hbm, ones_v)
    plsc.subcore_barrier()

    def body(j, carry):
        pltpu.sync_copy(ones_v, deg_sh.at[ridx_v.at[j]], add=True)
        return carry

    lax.fori_loop(0, CPW, body, 0)
    plsc.subcore_barrier()
    pltpu.sync_copy(deg_sh.at[pl.ds(base, _RPS)], out_hbm.at[c, pl.ds(base, _RPS)])


_NBUF = 2
_WIN = 40
_A0, _A1 = 160, 0


@functools.partial(
    pl.kernel,
    out_type=jax.ShapeDtypeStruct((2, NPAD, F), jnp.float32),
    mesh=_MESH,
    scratch_types=[
        pltpu.VMEM((_WIN, CHUNK), jnp.int32),
        pltpu.VMEM((_WIN, CHUNK), jnp.int32),
        pltpu.VMEM((_NBUF, CHUNK, F), jnp.float32),
        pltpu.VMEM_SHARED((NPAD, F), jnp.float32),
        [pltpu.SemaphoreType.DMA] * _NBUF,
    ],
)
def _sc_scatter(table_hbm, r_hbm, c_hbm, zeros_hbm, out_hbm,
                ridx_v, cidx_v, rows_nb, agg_sh, sems):
    rows_bufs = [rows_nb.at[b] for b in range(_NBUF)]
    c = lax.axis_index("c")
    s = lax.axis_index("s")
    base = s * _RPS
    pltpu.sync_copy(zeros_hbm.at[pl.ds(base, _RPS)], agg_sh.at[pl.ds(base, _RPS)])
    plsc.subcore_barrier()

    cnt = jnp.where(c == 0, _A0, _A1)
    off = c * (_NSUB * _A0) + s * cnt
    nwin = cnt // _WIN

    for win in range(max(_A0, _A1) // _WIN):
        @pl.when(win < nwin)
        def _(win=win):
            w0 = pl.multiple_of(off + win * _WIN, _WIN)
            pltpu.sync_copy(r_hbm.at[pl.ds(w0, _WIN)], ridx_v)
            pltpu.sync_copy(c_hbm.at[pl.ds(w0, _WIN)], cidx_v)
            for b in range(_NBUF):
                pltpu.async_copy(table_hbm.at[ridx_v.at[b]], rows_bufs[b], sems[b])

            def body(i, carry):
                j0 = i * _NBUF
                for b in range(_NBUF):
                    j = j0 + b
                    pltpu.make_async_copy(table_hbm.at[ridx_v.at[j]],
                                          rows_bufs[b], sems[b]).wait()
                    pltpu.sync_copy(rows_bufs[b], agg_sh.at[cidx_v.at[j]], add=True)
                    nxt = jnp.minimum(j + _NBUF, _WIN - 1)
                    pltpu.async_copy(table_hbm.at[ridx_v.at[nxt]], rows_bufs[b], sems[b])
                return carry

            lax.fori_loop(0, _WIN // _NBUF, body, 0)
            for b in range(_NBUF):
                pltpu.make_async_copy(table_hbm.at[ridx_v.at[_WIN - 1]],
                                      rows_bufs[b], sems[b]).wait()
    plsc.subcore_barrier()
    pltpu.sync_copy(agg_sh.at[pl.ds(base, _RPS)], out_hbm.at[c, pl.ds(base, _RPS)])


def kernel(x, edge_index, batch, bn_feat_g, bn_feat_b, W_feat,
           bnc_g0, bnc_b0, Wc0, bc0, bnc_g1, bnc_b1, Wc1, bc1,
           bnc_g2, bnc_b2, Wc2, bc2, bn_fc_g, bn_fc_b, W_fc, b_fc,
           bn_hid_g, bn_hid_b, W_cls, b_cls):
    f32 = jnp.float32
    r2 = lambda a: a.reshape(1, -1).astype(f32)
    g_feat, b_feat = r2(bn_feat_g), r2(bn_feat_b)
    g0, b0, g1, b1, g2, b2 = map(r2, (bnc_g0, bnc_b0, bnc_g1, bnc_b1, bnc_g2, bnc_b2))
    bc0r, bc1r, bc2r = map(r2, (bc0, bc1, bc2))
    fcg, fcb, bfc, hg, hb, bcls = map(r2, (bn_fc_g, bn_fc_b, b_fc, bn_hid_g, bn_hid_b, b_cls))
    batch2d = batch.reshape(1, N)

    pad = jnp.full((EPAD - E,), N, jnp.int32)
    rp = jnp.concatenate([edge_index[0], pad]).reshape(NW * CPW, CHUNK)
    cp = jnp.concatenate([edge_index[1], pad]).reshape(NW * CPW, CHUNK)
    zeros_f = jnp.zeros((NPAD, F), f32)
    ones_f = jnp.ones((CHUNK, F), f32)

    degp = _sc_deg(rp, ones_f, zeros_f)

    tab = _tc_call(_tc_feat_body, jax.ShapeDtypeStruct((NPAD, F), f32),
                   x, g_feat, b_feat, W_feat, g0, b0, Wc0, degp)

    for Wn, gn, bn_, bcr in ((Wc1, g1, b1, bc0r), (Wc2, g2, b2, bc1r)):
        parts = _sc_scatter(tab, rp, cp, zeros_f)
        tab = _tc_call(_tc_combine_body, jax.ShapeDtypeStruct((NPAD, F), f32),
                       parts, tab, degp, bcr, gn, bn_, Wn)

    parts = _sc_scatter(tab, rp, cp, zeros_f)
    out = _tc_call(_tc_head_body, jax.ShapeDtypeStruct((NG, NCLS), f32),
                   parts, tab, degp, bc2r, batch2d, fcg, fcb, W_fc, bfc,
                   hg, hb, W_cls, bcls)
    return out


# speedup vs baseline: 1.0736x; 1.0736x over previous
import functools

import jax
import jax.numpy as jnp
from jax import lax
from jax.experimental import pallas as pl
from jax.experimental.pallas import tpu as pltpu
from jax.experimental.pallas import tpu_sc as plsc

N = 10000
NPAD = 10112
F = 128
NG = 64
NCLS = 10
EPS = 1e-5
E = 320000
NW = 32
CHUNK = 128
CPW = 80
EPAD = NW * CPW * CHUNK


def _bn(x, g, b):
    m = jnp.mean(x, axis=0, keepdims=True)
    v = jnp.mean((x - m) ** 2, axis=0, keepdims=True)
    return g * (x - m) * lax.rsqrt(v + EPS) + b


def _dis_from_parts(degp_ref):
    deg = degp_ref[0, :N, :1] + degp_ref[1, :N, :1]
    return lax.rsqrt(deg + 1.0)


def _tc_feat_body(x_ref, g_ref, b_ref, Wf_ref, g0_ref, b0_ref, Wc0_ref,
                  degp_ref, tab0_ref):
    h = _bn(x_ref[...], g_ref[...], b_ref[...])
    h = jnp.maximum(jnp.dot(h, Wf_ref[...], preferred_element_type=jnp.float32), 0.0)
    t0 = jnp.dot(_bn(h, g0_ref[...], b0_ref[...]), Wc0_ref[...],
                 preferred_element_type=jnp.float32)
    dis = _dis_from_parts(degp_ref)
    tab0_ref[:N, :] = dis * t0
    tab0_ref[N:, :] = jnp.zeros((NPAD - N, F), jnp.float32)


def _tc_combine_body(parts_ref, tab_ref, degp_ref, bc_ref, g_ref, b_ref,
                     W_ref, out_ref):
    dis = _dis_from_parts(degp_ref)
    S = parts_ref[0, :N, :] + parts_ref[1, :N, :]
    h = jnp.maximum(dis * (S + tab_ref[:N, :]) + bc_ref[...], 0.0)
    t = jnp.dot(_bn(h, g_ref[...], b_ref[...]), W_ref[...],
                preferred_element_type=jnp.float32)
    out_ref[:N, :] = dis * t
    out_ref[N:, :] = jnp.zeros((NPAD - N, F), jnp.float32)


def _tc_head_body(parts_ref, tab_ref, degp_ref, bc_ref, batch_ref,
                  fcg_ref, fcb_ref, Wfc_ref, bfc_ref, hg_ref, hb_ref,
                  Wcls_ref, bcls_ref, out_ref):
    dis = _dis_from_parts(degp_ref)
    S = parts_ref[0, :N, :] + parts_ref[1, :N, :]
    h = jnp.maximum(dis * (S + tab_ref[:N, :]) + bc_ref[...], 0.0)
    oh = (lax.broadcasted_iota(jnp.int32, (NG, N), 0) == batch_ref[...]
          ).astype(jnp.float32)
    pooled = jnp.dot(oh, h, preferred_element_type=jnp.float32)
    p = _bn(pooled, fcg_ref[...], fcb_ref[...])
    p = jnp.maximum(jnp.dot(p, Wfc_ref[...], preferred_element_type=jnp.float32)
                    + bfc_ref[...], 0.0)
    p = _bn(p, hg_ref[...], hb_ref[...])
    logits = jnp.dot(p, Wcls_ref[...], preferred_element_type=jnp.float32) + bcls_ref[...]
    m = jnp.max(logits, axis=-1, keepdims=True)
    lse = jnp.log(jnp.sum(jnp.exp(logits - m), axis=-1, keepdims=True))
    out_ref[...] = (logits - m) - lse


def _tc_call(body, out_shape, *args):
    return pl.pallas_call(body, out_shape=out_shape)(*args)


_MESH = plsc.VectorSubcoreMesh(core_axis_name="c", subcore_axis_name="s")
_NSUB = 16
_RPS = NPAD // _NSUB


@functools.partial(
    pl.kernel,
    out_type=jax.ShapeDtypeStruct((2, NPAD, F), jnp.float32),
    mesh=_MESH,
    scratch_types=[
        pltpu.VMEM((CPW, CHUNK), jnp.int32),
        pltpu.VMEM((CHUNK, F), jnp.float32),
        pltpu.VMEM_SHARED((NPAD, F), jnp.float32),
    ],
)
def _sc_deg(r_hbm, ones_hbm, zeros_hbm, out_hbm, ridx_v, ones_v, deg_sh):
    c = lax.axis_index("c")
    s = lax.axis_index("s")
    w = c * _NSUB + s
    base = s * _RPS
    pltpu.sync_copy(zeros_hbm.at[pl.ds(base, _RPS)], deg_sh.at[pl.ds(base, _RPS)])
    pltpu.sync_copy(r_hbm.at[pl.ds(pl.multiple_of(w * CPW, CPW), CPW)], ridx_v)
    pltpu.sync_copy(ones_hbm, ones_v)
    plsc.subcore_barrier()

    def body(j, carry):
        pltpu.sync_copy(ones_v, deg_sh.at[ridx_v.at[j]], add=True)
        return carry

    lax.fori_loop(0, CPW, body, 0)
    plsc.subcore_barrier()
    pltpu.sync_copy(deg_sh.at[pl.ds(base, _RPS)], out_hbm.at[c, pl.ds(base, _RPS)])


_NBUF = 2
_WIN = 16
_A0, _A1 = 112, 48


@functools.partial(
    pl.kernel,
    out_type=jax.ShapeDtypeStruct((2, NPAD, F), jnp.float32),
    mesh=_MESH,
    scratch_types=[
        pltpu.VMEM((_WIN, CHUNK), jnp.int32),
        pltpu.VMEM((_WIN, CHUNK), jnp.int32),
        pltpu.VMEM((_NBUF, CHUNK, F), jnp.float32),
        pltpu.VMEM_SHARED((NPAD, F), jnp.float32),
        [pltpu.SemaphoreType.DMA] * _NBUF,
    ],
)
def _sc_scatter(table_hbm, r_hbm, c_hbm, zeros_hbm, out_hbm,
                ridx_v, cidx_v, rows_nb, agg_sh, sems):
    rows_bufs = [rows_nb.at[b] for b in range(_NBUF)]
    c = lax.axis_index("c")
    s = lax.axis_index("s")
    base = s * _RPS
    pltpu.sync_copy(zeros_hbm.at[pl.ds(base, _RPS)], agg_sh.at[pl.ds(base, _RPS)])
    plsc.subcore_barrier()

    cnt = jnp.where(c == 0, _A0, _A1)
    off = c * (_NSUB * _A0) + s * cnt
    nwin = cnt // _WIN

    for win in range(max(_A0, _A1) // _WIN):
        @pl.when(win < nwin)
        def _(win=win):
            w0 = pl.multiple_of(off + win * _WIN, _WIN)
            pltpu.sync_copy(r_hbm.at[pl.ds(w0, _WIN)], ridx_v)
            pltpu.sync_copy(c_hbm.at[pl.ds(w0, _WIN)], cidx_v)
            for b in range(_NBUF):
                pltpu.async_copy(table_hbm.at[ridx_v.at[b]], rows_bufs[b], sems[b])

            def body(i, carry):
                j0 = i * _NBUF
                for b in range(_NBUF):
                    j = j0 + b
                    pltpu.make_async_copy(table_hbm.at[ridx_v.at[j]],
                                          rows_bufs[b], sems[b]).wait()
                    pltpu.sync_copy(rows_bufs[b], agg_sh.at[cidx_v.at[j]], add=True)
                    nxt = jnp.minimum(j + _NBUF, _WIN - 1)
                    pltpu.async_copy(table_hbm.at[ridx_v.at[nxt]], rows_bufs[b], sems[b])
                return carry

            lax.fori_loop(0, _WIN // _NBUF, body, 0)
            for b in range(_NBUF):
                pltpu.make_async_copy(table_hbm.at[ridx_v.at[_WIN - 1]],
                                      rows_bufs[b], sems[b]).wait()
    plsc.subcore_barrier()
    pltpu.sync_copy(agg_sh.at[pl.ds(base, _RPS)], out_hbm.at[c, pl.ds(base, _RPS)])


def kernel(x, edge_index, batch, bn_feat_g, bn_feat_b, W_feat,
           bnc_g0, bnc_b0, Wc0, bc0, bnc_g1, bnc_b1, Wc1, bc1,
           bnc_g2, bnc_b2, Wc2, bc2, bn_fc_g, bn_fc_b, W_fc, b_fc,
           bn_hid_g, bn_hid_b, W_cls, b_cls):
    f32 = jnp.float32
    r2 = lambda a: a.reshape(1, -1).astype(f32)
    g_feat, b_feat = r2(bn_feat_g), r2(bn_feat_b)
    g0, b0, g1, b1, g2, b2 = map(r2, (bnc_g0, bnc_b0, bnc_g1, bnc_b1, bnc_g2, bnc_b2))
    bc0r, bc1r, bc2r = map(r2, (bc0, bc1, bc2))
    fcg, fcb, bfc, hg, hb, bcls = map(r2, (bn_fc_g, bn_fc_b, b_fc, bn_hid_g, bn_hid_b, b_cls))
    batch2d = batch.reshape(1, N)

    pad = jnp.full((EPAD - E,), N, jnp.int32)
    rp = jnp.concatenate([edge_index[0], pad]).reshape(NW * CPW, CHUNK)
    cp = jnp.concatenate([edge_index[1], pad]).reshape(NW * CPW, CHUNK)
    zeros_f = jnp.zeros((NPAD, F), f32)
    ones_f = jnp.ones((CHUNK, F), f32)

    degp = _sc_deg(rp, ones_f, zeros_f)

    tab = _tc_call(_tc_feat_body, jax.ShapeDtypeStruct((NPAD, F), f32),
                   x, g_feat, b_feat, W_feat, g0, b0, Wc0, degp)

    for Wn, gn, bn_, bcr in ((Wc1, g1, b1, bc0r), (Wc2, g2, b2, bc1r)):
        parts = _sc_scatter(tab, rp, cp, zeros_f)
        tab = _tc_call(_tc_combine_body, jax.ShapeDtypeStruct((NPAD, F), f32),
                       parts, tab, degp, bcr, gn, bn_, Wn)

    parts = _sc_scatter(tab, rp, cp, zeros_f)
    out = _tc_call(_tc_head_body, jax.ShapeDtypeStruct((NG, NCLS), f32),
                   parts, tab, degp, bc2r, batch2d, fcg, fcb, W_fc, bfc,
                   hg, hb, W_cls, bcls)
    return out


# speedup vs baseline: 1.2012x; 1.1188x over previous
import functools

import jax
import jax.numpy as jnp
from jax import lax
from jax.experimental import pallas as pl
from jax.experimental.pallas import tpu as pltpu
from jax.experimental.pallas import tpu_sc as plsc

N = 10000
NPAD = 10112
F = 128
NG = 64
NCLS = 10
EPS = 1e-5
E = 320000
NW = 32
CHUNK = 128
CPW = 80
EPAD = NW * CPW * CHUNK


def _bn(x, g, b):
    m = jnp.mean(x, axis=0, keepdims=True)
    v = jnp.mean((x - m) ** 2, axis=0, keepdims=True)
    return g * (x - m) * lax.rsqrt(v + EPS) + b


def _dis_from_parts(degp_ref):
    deg = degp_ref[0, :N, :1] + degp_ref[1, :N, :1]
    return lax.rsqrt(deg + 1.0)


def _tc_feat_body(x_ref, g_ref, b_ref, Wf_ref, g0_ref, b0_ref, Wc0_ref,
                  degp_ref, tab0_ref):
    h = _bn(x_ref[...], g_ref[...], b_ref[...])
    h = jnp.maximum(jnp.dot(h, Wf_ref[...], preferred_element_type=jnp.float32), 0.0)
    t0 = jnp.dot(_bn(h, g0_ref[...], b0_ref[...]), Wc0_ref[...],
                 preferred_element_type=jnp.float32)
    dis = _dis_from_parts(degp_ref)
    tab0_ref[:N, :] = dis * t0
    tab0_ref[N:, :] = jnp.zeros((NPAD - N, F), jnp.float32)


def _tc_combine_body(parts_ref, tab_ref, degp_ref, bc_ref, g_ref, b_ref,
                     W_ref, out_ref):
    dis = _dis_from_parts(degp_ref)
    S = parts_ref[0, :N, :] + parts_ref[1, :N, :]
    h = jnp.maximum(dis * (S + tab_ref[:N, :]) + bc_ref[...], 0.0)
    t = jnp.dot(_bn(h, g_ref[...], b_ref[...]), W_ref[...],
                preferred_element_type=jnp.float32)
    out_ref[:N, :] = dis * t
    out_ref[N:, :] = jnp.zeros((NPAD - N, F), jnp.float32)


def _tc_head_body(parts_ref, tab_ref, degp_ref, bc_ref, batch_ref,
                  fcg_ref, fcb_ref, Wfc_ref, bfc_ref, hg_ref, hb_ref,
                  Wcls_ref, bcls_ref, out_ref):
    dis = _dis_from_parts(degp_ref)
    S = parts_ref[0, :N, :] + parts_ref[1, :N, :]
    h = jnp.maximum(dis * (S + tab_ref[:N, :]) + bc_ref[...], 0.0)
    oh = (lax.broadcasted_iota(jnp.int32, (NG, N), 0) == batch_ref[...]
          ).astype(jnp.float32)
    pooled = jnp.dot(oh, h, preferred_element_type=jnp.float32)
    p = _bn(pooled, fcg_ref[...], fcb_ref[...])
    p = jnp.maximum(jnp.dot(p, Wfc_ref[...], preferred_element_type=jnp.float32)
                    + bfc_ref[...], 0.0)
    p = _bn(p, hg_ref[...], hb_ref[...])
    logits = jnp.dot(p, Wcls_ref[...], preferred_element_type=jnp.float32) + bcls_ref[...]
    m = jnp.max(logits, axis=-1, keepdims=True)
    lse = jnp.log(jnp.sum(jnp.exp(logits - m), axis=-1, keepdims=True))
    out_ref[...] = (logits - m) - lse


def _tc_call(body, out_shape, *args):
    return pl.pallas_call(body, out_shape=out_shape)(*args)


_MESH = plsc.VectorSubcoreMesh(core_axis_name="c", subcore_axis_name="s")
_NSUB = 16
_RPS = NPAD // _NSUB


@functools.partial(
    pl.kernel,
    out_type=jax.ShapeDtypeStruct((2, NPAD, F), jnp.float32),
    mesh=_MESH,
    scratch_types=[
        pltpu.VMEM((CPW, CHUNK), jnp.int32),
        pltpu.VMEM((CHUNK, F), jnp.float32),
        pltpu.VMEM_SHARED((NPAD, F), jnp.float32),
    ],
)
def _sc_deg(r_hbm, ones_hbm, zeros_hbm, out_hbm, ridx_v, ones_v, deg_sh):
    c = lax.axis_index("c")
    s = lax.axis_index("s")
    w = c * _NSUB + s
    base = s * _RPS
    pltpu.sync_copy(zeros_hbm.at[pl.ds(base, _RPS)], deg_sh.at[pl.ds(base, _RPS)])
    pltpu.sync_copy(r_hbm.at[pl.ds(pl.multiple_of(w * CPW, CPW), CPW)], ridx_v)
    pltpu.sync_copy(ones_hbm, ones_v)
    plsc.subcore_barrier()

    def body(j, carry):
        pltpu.sync_copy(ones_v, deg_sh.at[ridx_v.at[j]], add=True)
        return carry

    lax.fori_loop(0, CPW, body, 0)
    plsc.subcore_barrier()
    pltpu.sync_copy(deg_sh.at[pl.ds(base, _RPS)], out_hbm.at[c, pl.ds(base, _RPS)])


_NBUF = 2
_WIN = 16
_A0, _A1 = 144, 16


@functools.partial(
    pl.kernel,
    out_type=jax.ShapeDtypeStruct((2, NPAD, F), jnp.float32),
    mesh=_MESH,
    scratch_types=[
        pltpu.VMEM((_WIN, CHUNK), jnp.int32),
        pltpu.VMEM((_WIN, CHUNK), jnp.int32),
        pltpu.VMEM((_NBUF, CHUNK, F), jnp.float32),
        pltpu.VMEM_SHARED((NPAD, F), jnp.float32),
        [pltpu.SemaphoreType.DMA] * _NBUF,
    ],
)
def _sc_scatter(table_hbm, r_hbm, c_hbm, zeros_hbm, out_hbm,
                ridx_v, cidx_v, rows_nb, agg_sh, sems):
    rows_bufs = [rows_nb.at[b] for b in range(_NBUF)]
    c = lax.axis_index("c")
    s = lax.axis_index("s")
    base = s * _RPS
    pltpu.sync_copy(zeros_hbm.at[pl.ds(base, _RPS)], agg_sh.at[pl.ds(base, _RPS)])
    plsc.subcore_barrier()

    cnt = jnp.where(c == 0, _A0, _A1)
    off = c * (_NSUB * _A0) + s * cnt
    nwin = cnt // _WIN

    for win in range(max(_A0, _A1) // _WIN):
        @pl.when(win < nwin)
        def _(win=win):
            w0 = pl.multiple_of(off + win * _WIN, _WIN)
            pltpu.sync_copy(r_hbm.at[pl.ds(w0, _WIN)], ridx_v)
            pltpu.sync_copy(c_hbm.at[pl.ds(w0, _WIN)], cidx_v)
            for b in range(_NBUF):
                pltpu.async_copy(table_hbm.at[ridx_v.at[b]], rows_bufs[b], sems[b])

            def body(i, carry):
                j0 = i * _NBUF
                for b in range(_NBUF):
                    j = j0 + b
                    pltpu.make_async_copy(table_hbm.at[ridx_v.at[j]],
                                          rows_bufs[b], sems[b]).wait()
                    pltpu.sync_copy(rows_bufs[b], agg_sh.at[cidx_v.at[j]], add=True)
                    nxt = jnp.minimum(j + _NBUF, _WIN - 1)
                    pltpu.async_copy(table_hbm.at[ridx_v.at[nxt]], rows_bufs[b], sems[b])
                return carry

            lax.fori_loop(0, _WIN // _NBUF, body, 0)
            for b in range(_NBUF):
                pltpu.make_async_copy(table_hbm.at[ridx_v.at[_WIN - 1]],
                                      rows_bufs[b], sems[b]).wait()
    plsc.subcore_barrier()
    pltpu.sync_copy(agg_sh.at[pl.ds(base, _RPS)], out_hbm.at[c, pl.ds(base, _RPS)])


def kernel(x, edge_index, batch, bn_feat_g, bn_feat_b, W_feat,
           bnc_g0, bnc_b0, Wc0, bc0, bnc_g1, bnc_b1, Wc1, bc1,
           bnc_g2, bnc_b2, Wc2, bc2, bn_fc_g, bn_fc_b, W_fc, b_fc,
           bn_hid_g, bn_hid_b, W_cls, b_cls):
    f32 = jnp.float32
    r2 = lambda a: a.reshape(1, -1).astype(f32)
    g_feat, b_feat = r2(bn_feat_g), r2(bn_feat_b)
    g0, b0, g1, b1, g2, b2 = map(r2, (bnc_g0, bnc_b0, bnc_g1, bnc_b1, bnc_g2, bnc_b2))
    bc0r, bc1r, bc2r = map(r2, (bc0, bc1, bc2))
    fcg, fcb, bfc, hg, hb, bcls = map(r2, (bn_fc_g, bn_fc_b, b_fc, bn_hid_g, bn_hid_b, b_cls))
    batch2d = batch.reshape(1, N)

    pad = jnp.full((EPAD - E,), N, jnp.int32)
    rp = jnp.concatenate([edge_index[0], pad]).reshape(NW * CPW, CHUNK)
    cp = jnp.concatenate([edge_index[1], pad]).reshape(NW * CPW, CHUNK)
    zeros_f = jnp.zeros((NPAD, F), f32)
    ones_f = jnp.ones((CHUNK, F), f32)

    degp = _sc_deg(rp, ones_f, zeros_f)

    tab = _tc_call(_tc_feat_body, jax.ShapeDtypeStruct((NPAD, F), f32),
                   x, g_feat, b_feat, W_feat, g0, b0, Wc0, degp)

    for Wn, gn, bn_, bcr in ((Wc1, g1, b1, bc0r), (Wc2, g2, b2, bc1r)):
        parts = _sc_scatter(tab, rp, cp, zeros_f)
        tab = _tc_call(_tc_combine_body, jax.ShapeDtypeStruct((NPAD, F), f32),
                       parts, tab, degp, bcr, gn, bn_, Wn)

    parts = _sc_scatter(tab, rp, cp, zeros_f)
    out = _tc_call(_tc_head_body, jax.ShapeDtypeStruct((NG, NCLS), f32),
                   parts, tab, degp, bc2r, batch2d, fcg, fcb, W_fc, bfc,
                   hg, hb, W_cls, bcls)
    return out
